# baseline (speedup 1.0000x reference)
import functools

import jax
import jax.numpy as jnp
from jax import lax
from jax.experimental import pallas as pl
from jax.experimental.pallas import tpu as pltpu
from jax.experimental.pallas import tpu_sc as plsc

N = 10000
E = 320000
D = 128

NC, NS = 2, 16
NW = NC * NS
EPW = E // NW
CH = 128
EPWP = 10240
NCH = EPWP // CH
PAD = EPWP - EPW
NP = 10240
RPT = NP // NS

_MESH = plsc.VectorSubcoreMesh(core_axis_name="c", subcore_axis_name="s")


NBUF = 2


def _make_sc_agg(with_counts):
    out_type = [jax.ShapeDtypeStruct((NC, NP, D), jnp.float32)]
    scratch = [
        pltpu.VMEM((EPWP,), jnp.int32),
        pltpu.VMEM((NBUF, CH), jnp.int32),
        pltpu.VMEM((NBUF * CH, D), jnp.float32),
        pltpu.VMEM_SHARED((NP, D), jnp.float32),
    ] + [pltpu.SemaphoreType.DMA] * (3 * NBUF)
    if with_counts:
        out_type.append(jax.ShapeDtypeStruct((NC, NP), jnp.float32))
        scratch += [
            pltpu.VMEM((CH,), jnp.float32),
            pltpu.VMEM_SHARED((NP,), jnp.float32),
        ] + [pltpu.SemaphoreType.DMA] * NBUF

    def agg(*args):
        it = iter(args)
        h_hbm, src_hbm, dst_hbm, z_hbm = (next(it) for _ in range(4))
        z1_hbm = next(it) if with_counts else None
        out_hbm = next(it)
        cnt_hbm = next(it) if with_counts else None
        sidx, dbufs, rbufs, acc = (next(it) for _ in range(4))
        dsems = [next(it) for _ in range(NBUF)]
        gsems = [next(it) for _ in range(NBUF)]
        ssems = [next(it) for _ in range(NBUF)]
        if with_counts:
            ones_v, acc1 = next(it), next(it)
            csems = [next(it) for _ in range(NBUF)]

        c = lax.axis_index("c")
        s = lax.axis_index("s")
        base = (s * NC + c) * EPWP
        r0 = s * RPT

        pltpu.sync_copy(src_hbm.at[pl.ds(base, EPWP)], sidx)
        pltpu.sync_copy(z_hbm.at[pl.ds(r0, RPT)], acc.at[pl.ds(r0, RPT)])
        if with_counts:
            for k in range(CH // 16):
                ones_v[pl.ds(16 * k, 16)] = jnp.full((16,), 1.0, jnp.float32)
            pltpu.sync_copy(z1_hbm.at[pl.ds(r0, RPT)], acc1.at[pl.ds(r0, RPT)])
        plsc.subcore_barrier()

        def rbuf(b):
            return rbufs.at[pl.ds(b * CH, CH)]

        def fire(ci, b):
            pltpu.async_copy(dst_hbm.at[pl.ds(base + ci * CH, CH)],
                             dbufs.at[b], dsems[b])
            pltpu.async_copy(h_hbm.at[sidx.at[pl.ds(ci * CH, CH)]],
                             rbuf(b), gsems[b])

        def wait_in(b):
            pltpu.make_async_copy(dst_hbm.at[pl.ds(0, CH)], dbufs.at[b],
                                  dsems[b]).wait()
            pltpu.make_async_copy(h_hbm.at[pl.ds(0, CH)], rbuf(b),
                                  gsems[b]).wait()

        def wait_scat(b):
            pltpu.make_async_copy(rbuf(b), acc.at[pl.ds(0, CH)],
                                  ssems[b]).wait()
            if with_counts:
                pltpu.make_async_copy(ones_v, acc1.at[pl.ds(0, CH)],
                                      csems[b]).wait()

        for b in range(NBUF - 1):
            fire(b, b)

        def body(j, carry):
            for b in range(NBUF):
                ci = NBUF * j + b
                wait_in(b)
                pltpu.async_copy(rbuf(b), acc.at[dbufs.at[b]], ssems[b],
                                 add=True)
                if with_counts:
                    pltpu.async_copy(ones_v, acc1.at[dbufs.at[b]], csems[b],
                                     add=True)
                nb = (b + NBUF - 1) % NBUF
                nc = ci + NBUF - 1

                @pl.when(jnp.logical_and(nc < NCH, nc >= NBUF))
                def _():
                    wait_scat(nb)
                    fire(nc, nb)

                @pl.when(jnp.logical_and(nc < NCH, nc < NBUF))
                def _():
                    fire(nc, nb)
            return carry

        lax.fori_loop(0, NCH // NBUF, body, 0)

        for b in range(NBUF):
            wait_scat(b)

        plsc.subcore_barrier()
        pltpu.sync_copy(acc.at[pl.ds(r0, RPT)], out_hbm.at[c, pl.ds(r0, RPT)])
        if with_counts:
            pltpu.sync_copy(acc1.at[pl.ds(r0, RPT)],
                            cnt_hbm.at[c, pl.ds(r0, RPT)])

    return pl.kernel(agg, mesh=_MESH, out_type=out_type,
                     scratch_types=scratch)


_AGG1 = _make_sc_agg(with_counts=True)
_AGG = _make_sc_agg(with_counts=False)


BN = 1000


def _tc1_body(p0_ref, p1_ref, c0_ref, c1_ref, x_ref, wl_ref, b_ref, wr_ref,
              h_ref, rc_ref):
    cnt = c0_ref[...] + c1_ref[...]
    rc = 1.0 / jnp.maximum(cnt, 1.0)
    mean = (p0_ref[0] + p1_ref[0]) * rc
    acc = jnp.dot(mean, wl_ref[...], preferred_element_type=jnp.float32)
    acc = acc + jnp.dot(x_ref[...], wr_ref[...], preferred_element_type=jnp.float32)
    acc = acc + b_ref[...]
    h_ref[...] = jnp.maximum(acc, 0.0)
    rc_ref[...] = rc


def _tc_layer1(p, c0, c1, x, Wl, b, Wr):
    return pl.pallas_call(
        _tc1_body,
        grid=(N // BN,),
        in_specs=[
            pl.BlockSpec((1, BN, D), lambda i: (0, i, 0)),
            pl.BlockSpec((1, BN, D), lambda i: (1, i, 0)),
            pl.BlockSpec((BN, 1), lambda i: (i, 0)),
            pl.BlockSpec((BN, 1), lambda i: (i, 0)),
            pl.BlockSpec((BN, D), lambda i: (i, 0)),
            pl.BlockSpec((D, D), lambda i: (0, 0)),
            pl.BlockSpec((1, D), lambda i: (0, 0)),
            pl.BlockSpec((D, D), lambda i: (0, 0)),
        ],
        out_specs=[
            pl.BlockSpec((BN, D), lambda i: (i, 0)),
            pl.BlockSpec((BN, 1), lambda i: (i, 0)),
        ],
        out_shape=[
            jax.ShapeDtypeStruct((N, D), jnp.float32),
            jax.ShapeDtypeStruct((N, 1), jnp.float32),
        ],
    )(p, p, c0, c1, x, Wl, b, Wr)


def _make_tc23_body(relu):
    def body(p0_ref, p1_ref, x_ref, rc_ref, wl_ref, b_ref, wr_ref, h_ref):
        mean = (p0_ref[0] + p1_ref[0]) * rc_ref[...]
        acc = jnp.dot(mean, wl_ref[...], preferred_element_type=jnp.float32)
        acc = acc + jnp.dot(x_ref[...], wr_ref[...], preferred_element_type=jnp.float32)
        acc = acc + b_ref[...]
        h_ref[...] = jnp.maximum(acc, 0.0) if relu else acc
    return body


def _tc_layer23(p, x, rc, Wl, b, Wr, relu):
    return pl.pallas_call(
        _make_tc23_body(relu),
        grid=(N // BN,),
        in_specs=[
            pl.BlockSpec((1, BN, D), lambda i: (0, i, 0)),
            pl.BlockSpec((1, BN, D), lambda i: (1, i, 0)),
            pl.BlockSpec((BN, D), lambda i: (i, 0)),
            pl.BlockSpec((BN, 1), lambda i: (i, 0)),
            pl.BlockSpec((D, D), lambda i: (0, 0)),
            pl.BlockSpec((1, D), lambda i: (0, 0)),
            pl.BlockSpec((D, D), lambda i: (0, 0)),
        ],
        out_specs=pl.BlockSpec((BN, D), lambda i: (i, 0)),
        out_shape=jax.ShapeDtypeStruct((N, D), jnp.float32),
    )(p, p, x, rc, Wl, b, Wr)


def _pad_edges(src, dst):
    srcw = src.reshape(NW, EPW)
    dstw = dst.reshape(NW, EPW)
    pad_ids = jnp.arange(NW * PAD, dtype=jnp.int32).reshape(NW, PAD)
    src_pad = pad_ids % N
    dst_pad = N + pad_ids % (NP - N)
    src_p = jnp.concatenate([srcw, src_pad], axis=1).reshape(-1)
    dst_p = jnp.concatenate([dstw, dst_pad], axis=1).reshape(-1)
    return src_p, dst_p


def kernel(x, edge_index, W1l, b1, W1r, W2l, b2, W2r, W3l, b3, W3r):
    src = edge_index[0].astype(jnp.int32)
    dst = edge_index[1].astype(jnp.int32)
    src_p, dst_p = _pad_edges(src, dst)

    z = jnp.zeros((NP, D), jnp.float32)
    z1 = jnp.zeros((NP,), jnp.float32)

    p, cp = _AGG1(x, src_p, dst_p, z, z1)
    h1, rc = _tc_layer1(p, cp[0, :N].reshape(N, 1), cp[1, :N].reshape(N, 1),
                        x, W1l, b1.reshape(1, D), W1r)

    p, = _AGG(h1, src_p, dst_p, z)
    h2 = _tc_layer23(p, h1, rc, W2l, b2.reshape(1, D), W2r, relu=True)

    p, = _AGG(h2, src_p, dst_p, z)
    h3 = _tc_layer23(p, h2, rc, W3l, b3.reshape(1, D), W3r, relu=False)
    return h3
